# Initial kernel scaffold; baseline (speedup 1.0000x reference)
import jax
import jax.numpy as jnp
from jax.experimental import pallas as pl


def kernel(grads, labels, centroids, initialized):
    raise NotImplementedError("write your pallas kernel here")


# speedup vs baseline: 4.8203x; 4.8203x over previous
import functools

import jax
import jax.numpy as jnp
from jax import lax
from jax.experimental import pallas as pl
from jax.experimental.pallas import tpu as pltpu
from jax.experimental.pallas import tpu_sc as plsc

N = 320000
DIM = 128
NUM_CLASSES = 1000
CP = 1024
MOMENTUM = 0.9

NC = 2
NS = 16
NW = NC * NS
BLK = 128
NBLK = N // BLK
BASE_NBLK = NBLK // NW
REM = NBLK - BASE_NBLK * NW
ROWS_PER_TILE = CP // NS


def _seg_body(grads_hbm, labels_hbm, sums_hbm, counts_hbm,
              rows_v, lab_v, ones_v, zrow_v, acc_sh, cacc_sh):
    c = lax.axis_index("c")
    s = lax.axis_index("s")
    w = c * NS + s

    zero16 = jnp.zeros((16,), jnp.float32)
    one16 = jnp.ones((16,), jnp.float32)

    def fill_zero(r, _):
        for j in range(DIM // 16):
            zrow_v[r, pl.ds(j * 16, 16)] = zero16
        return 0

    lax.fori_loop(0, ROWS_PER_TILE, fill_zero, 0)

    def fill_ones(r, _):
        for j in range(DIM // 16):
            ones_v[r, pl.ds(j * 16, 16)] = one16
        return 0

    lax.fori_loop(0, BLK, fill_ones, 0)

    pltpu.sync_copy(zrow_v, acc_sh.at[pl.ds(s * ROWS_PER_TILE, ROWS_PER_TILE)])
    pltpu.sync_copy(zrow_v, cacc_sh.at[pl.ds(s * ROWS_PER_TILE, ROWS_PER_TILE)])
    plsc.subcore_barrier()

    nblk = BASE_NBLK + jnp.where(w < REM, 1, 0)
    base_blk = w * BASE_NBLK + jnp.minimum(w, REM)

    def body(i, _):
        row0 = (base_blk + i) * BLK
        pltpu.sync_copy(grads_hbm.at[pl.ds(row0, BLK)], rows_v)
        pltpu.sync_copy(labels_hbm.at[pl.ds(row0, BLK)], lab_v.at[0])
        pltpu.sync_copy(rows_v, acc_sh.at[lab_v.at[0]], add=True)
        pltpu.sync_copy(ones_v, cacc_sh.at[lab_v.at[0]], add=True)
        return 0

    lax.fori_loop(0, nblk, body, 0)

    plsc.subcore_barrier()
    pltpu.sync_copy(acc_sh.at[pl.ds(s * ROWS_PER_TILE, ROWS_PER_TILE)],
                    sums_hbm.at[c, pl.ds(s * ROWS_PER_TILE, ROWS_PER_TILE)])
    pltpu.sync_copy(cacc_sh.at[pl.ds(s * ROWS_PER_TILE, ROWS_PER_TILE)],
                    counts_hbm.at[c, pl.ds(s * ROWS_PER_TILE, ROWS_PER_TILE)])


@functools.cache
def _build_seg_sum():
  return functools.partial(
    pl.kernel,
    out_type=(jax.ShapeDtypeStruct((NC, CP, DIM), jnp.float32),
              jax.ShapeDtypeStruct((NC, CP, DIM), jnp.float32)),
    mesh=plsc.VectorSubcoreMesh(core_axis_name="c", subcore_axis_name="s",
                                num_cores=NC, num_subcores=NS),
    scratch_types=[
        pltpu.VMEM((BLK, DIM), jnp.float32),
        pltpu.VMEM((1, BLK), jnp.int32),
        pltpu.VMEM((BLK, DIM), jnp.float32),
        pltpu.VMEM((ROWS_PER_TILE, DIM), jnp.float32),
        pltpu.VMEM_SHARED((CP, DIM), jnp.float32),
        pltpu.VMEM_SHARED((CP, DIM), jnp.float32),
    ],
  )(_seg_body)


def _v_body(sums_ref, counts_ref, cent_ref, init_ref, v_ref):
    sums = sums_ref[0] + sums_ref[1]
    counts = counts_ref[0, :, 0:1] + counts_ref[1, :, 0:1]
    means = sums / jnp.maximum(counts, 1.0)
    cent = cent_ref[...]
    init = init_ref[0, 0]
    ema = MOMENTUM * cent + (1.0 - MOMENTUM) * means
    upd = jnp.where(init > 0.5, ema, means)
    newc = jnp.where(counts > 0.0, upd, cent)
    rowid = lax.broadcasted_iota(jnp.int32, (CP, 1), 0)
    valid = rowid < NUM_CLASSES
    newc = jnp.where(valid, newc, 0.0)
    mean_c = jnp.sum(newc, axis=0, keepdims=True) * (1.0 / NUM_CLASSES)
    centered = jnp.where(valid, newc - mean_c, 0.0)

    gram = lax.dot_general(centered, centered, (((0,), (0,)), ((), ())),
                           preferred_element_type=jnp.float32)

    def _nrm(m):
        return m * lax.rsqrt(jnp.sum(m * m) + 1e-30)

    bmat = _nrm(gram)

    def _sq(_, m):
        return _nrm(jnp.dot(m, m, preferred_element_type=jnp.float32))

    bmat = lax.fori_loop(0, 12, _sq, bmat)

    ri = lax.broadcasted_iota(jnp.int32, (DIM, DIM), 0)
    ci = lax.broadcasted_iota(jnp.int32, (DIM, DIM), 1)
    diag = jnp.sum(jnp.where(ri == ci, bmat, 0.0), axis=1, keepdims=True)
    dmax = jnp.max(diag)
    v0 = jnp.sum(jnp.where(diag == dmax, bmat, 0.0), axis=0, keepdims=True)
    v0 = _nrm(v0)
    for _ in range(2):
        v0 = _nrm(jnp.dot(v0, gram, preferred_element_type=jnp.float32))
    v_ref[...] = v0


def _proj_body(g_ref, v_ref, o_ref):
    g = g_ref[...]
    v = v_ref[...]
    t = jnp.sum(g * v, axis=1, keepdims=True)
    o_ref[...] = g - t * v


PROJ_BR = 2560


def kernel(grads, labels, centroids, initialized):
    sums2, counts2 = _build_seg_sum()(grads, labels)

    cent_p = jnp.pad(centroids, ((0, CP - NUM_CLASSES), (0, 0)))
    init_f = initialized.astype(jnp.float32).reshape(1, 1)

    v = pl.pallas_call(
        _v_body,
        out_shape=jax.ShapeDtypeStruct((1, DIM), jnp.float32),
    )(sums2, counts2, cent_p, init_f)

    out = pl.pallas_call(
        _proj_body,
        grid=(N // PROJ_BR,),
        in_specs=[pl.BlockSpec((PROJ_BR, DIM), lambda i: (i, 0)),
                  pl.BlockSpec((1, DIM), lambda i: (0, 0))],
        out_specs=pl.BlockSpec((PROJ_BR, DIM), lambda i: (i, 0)),
        out_shape=jax.ShapeDtypeStruct((N, DIM), jnp.float32),
    )(grads, v)
    return out


# speedup vs baseline: 6.2906x; 1.3050x over previous
import functools

import jax
import jax.numpy as jnp
from jax import lax
from jax.experimental import pallas as pl
from jax.experimental.pallas import tpu as pltpu
from jax.experimental.pallas import tpu_sc as plsc

N = 320000
DIM = 128
NUM_CLASSES = 1000
CP = 1024
MOMENTUM = 0.9

NC = 2
NS = 16
NW = NC * NS
BLK = 128
CHUNK = 256
NCHUNK = N // CHUNK
NSLOT = -(-NCHUNK // NW)
NSLOT += NSLOT % 2
ROWS_PER_TILE = CP // NS


def _seg_body(grads_hbm, labels_hbm, sums_hbm, counts_hbm,
              rows_a, rows_b, labs_a, labs_b, ones_v, zrow_v, acc_sh, cacc_sh,
              sem_l0, sem_l1, sem_s0, sem_s1):
    c = lax.axis_index("c")
    s = lax.axis_index("s")
    w = c * NS + s
    rows = (rows_a, rows_b)
    labs = (labs_a, labs_b)
    sem_l = (sem_l0, sem_l1)
    sem_s = (sem_s0, sem_s1)

    zero16 = jnp.zeros((16,), jnp.float32)
    one16 = jnp.ones((16,), jnp.float32)

    def fill_zero(r, _):
        for j in range(DIM // 16):
            zrow_v[r, pl.ds(j * 16, 16)] = zero16
        return 0

    lax.fori_loop(0, ROWS_PER_TILE, fill_zero, 0)

    def fill_ones(r, _):
        for j in range(DIM // 16):
            ones_v[r, pl.ds(j * 16, 16)] = one16
        return 0

    lax.fori_loop(0, BLK, fill_ones, 0)

    def cid_of(slot):
        return w + slot * NW

    def start_load(slot, b):
        @pl.when(cid_of(slot) < NCHUNK)
        def _():
            cid = cid_of(slot)
            pltpu.async_copy(grads_hbm.at[pl.ds(cid * CHUNK, CHUNK)],
                             rows[b], sem_l[b])
            for j in range(CHUNK // BLK):
                pltpu.async_copy(
                    labels_hbm.at[pl.ds(cid * CHUNK + j * BLK, BLK)],
                    labs[b].at[j], sem_l[b])

    def wait_load(b):
        pltpu.make_async_copy(grads_hbm.at[pl.ds(0, CHUNK)],
                              rows[b], sem_l[b]).wait()
        for j in range(CHUNK // BLK):
            pltpu.make_async_copy(labels_hbm.at[pl.ds(0, BLK)],
                                  labs[b].at[j], sem_l[b]).wait()

    def fire_scatters(b):
        for j in range(CHUNK // BLK):
            pltpu.async_copy(rows[b].at[pl.ds(j * BLK, BLK)],
                             acc_sh.at[labs[b].at[j]], sem_s[b], add=True)
            pltpu.async_copy(ones_v, cacc_sh.at[labs[b].at[j]],
                             sem_s[b], add=True)

    def wait_scatters(b):
        for j in range(CHUNK // BLK):
            pltpu.make_async_copy(rows[b].at[pl.ds(j * BLK, BLK)],
                                  acc_sh.at[labs[b].at[j]], sem_s[b]).wait()
            pltpu.make_async_copy(ones_v, cacc_sh.at[labs[b].at[j]],
                                  sem_s[b]).wait()

    start_load(0, 0)

    pltpu.sync_copy(zrow_v, acc_sh.at[pl.ds(s * ROWS_PER_TILE, ROWS_PER_TILE)])
    pltpu.sync_copy(zrow_v, cacc_sh.at[pl.ds(s * ROWS_PER_TILE, ROWS_PER_TILE)])
    plsc.subcore_barrier()

    def step(slot, b):
        valid = cid_of(slot) < NCHUNK

        @pl.when(valid)
        def _():
            wait_load(b)
            fire_scatters(b)

        @pl.when(jnp.logical_and(slot >= 1, cid_of(slot - 1) < NCHUNK))
        def _():
            wait_scatters(1 - b)
        start_load(slot + 1, 1 - b)

    def pair(p, _):
        step(2 * p, 0)
        step(2 * p + 1, 1)
        return 0

    lax.fori_loop(0, NSLOT // 2, pair, 0)

    @pl.when(cid_of(NSLOT - 1) < NCHUNK)
    def _():
        wait_scatters((NSLOT - 1) % 2)

    plsc.subcore_barrier()
    pltpu.sync_copy(acc_sh.at[pl.ds(s * ROWS_PER_TILE, ROWS_PER_TILE)],
                    sums_hbm.at[c, pl.ds(s * ROWS_PER_TILE, ROWS_PER_TILE)])
    pltpu.sync_copy(cacc_sh.at[pl.ds(s * ROWS_PER_TILE, ROWS_PER_TILE)],
                    counts_hbm.at[c, pl.ds(s * ROWS_PER_TILE, ROWS_PER_TILE)])


@functools.cache
def _build_seg_sum():
  return functools.partial(
    pl.kernel,
    out_type=(jax.ShapeDtypeStruct((NC, CP, DIM), jnp.float32),
              jax.ShapeDtypeStruct((NC, CP, DIM), jnp.float32)),
    mesh=plsc.VectorSubcoreMesh(core_axis_name="c", subcore_axis_name="s",
                                num_cores=NC, num_subcores=NS),
    scratch_types=[
        pltpu.VMEM((CHUNK, DIM), jnp.float32),
        pltpu.VMEM((CHUNK, DIM), jnp.float32),
        pltpu.VMEM((CHUNK // BLK, BLK), jnp.int32),
        pltpu.VMEM((CHUNK // BLK, BLK), jnp.int32),
        pltpu.VMEM((BLK, DIM), jnp.float32),
        pltpu.VMEM((ROWS_PER_TILE, DIM), jnp.float32),
        pltpu.VMEM_SHARED((CP, DIM), jnp.float32),
        pltpu.VMEM_SHARED((CP, DIM), jnp.float32),
        pltpu.SemaphoreType.DMA,
        pltpu.SemaphoreType.DMA,
        pltpu.SemaphoreType.DMA,
        pltpu.SemaphoreType.DMA,
    ],
  )(_seg_body)


def _v_body(sums_ref, counts_ref, cent_ref, init_ref, v_ref):
    sums = sums_ref[0] + sums_ref[1]
    counts = counts_ref[0, :, 0:1] + counts_ref[1, :, 0:1]
    means = sums / jnp.maximum(counts, 1.0)
    cent = cent_ref[...]
    init = init_ref[0, 0]
    ema = MOMENTUM * cent + (1.0 - MOMENTUM) * means
    upd = jnp.where(init > 0.5, ema, means)
    newc = jnp.where(counts > 0.0, upd, cent)
    rowid = lax.broadcasted_iota(jnp.int32, (CP, 1), 0)
    valid = rowid < NUM_CLASSES
    newc = jnp.where(valid, newc, 0.0)
    mean_c = jnp.sum(newc, axis=0, keepdims=True) * (1.0 / NUM_CLASSES)
    centered = jnp.where(valid, newc - mean_c, 0.0)

    gram = lax.dot_general(centered, centered, (((0,), (0,)), ((), ())),
                           preferred_element_type=jnp.float32)

    def _nrm(m):
        return m * lax.rsqrt(jnp.sum(m * m) + 1e-30)

    bmat = _nrm(gram)

    def _sq(_, m):
        return _nrm(jnp.dot(m, m, preferred_element_type=jnp.float32))

    bmat = lax.fori_loop(0, 12, _sq, bmat)

    ri = lax.broadcasted_iota(jnp.int32, (DIM, DIM), 0)
    ci = lax.broadcasted_iota(jnp.int32, (DIM, DIM), 1)
    diag = jnp.sum(jnp.where(ri == ci, bmat, 0.0), axis=1, keepdims=True)
    dmax = jnp.max(diag)
    v0 = jnp.sum(jnp.where(diag == dmax, bmat, 0.0), axis=0, keepdims=True)
    v0 = _nrm(v0)
    for _ in range(2):
        v0 = _nrm(jnp.dot(v0, gram, preferred_element_type=jnp.float32))
    v_ref[...] = v0


def _proj_body(g_ref, v_ref, o_ref):
    g = g_ref[...]
    v = v_ref[...]
    t = jnp.sum(g * v, axis=1, keepdims=True)
    o_ref[...] = g - t * v


PROJ_BR = 2560


def kernel(grads, labels, centroids, initialized):
    sums2, counts2 = _build_seg_sum()(grads, labels)

    cent_p = jnp.pad(centroids, ((0, CP - NUM_CLASSES), (0, 0)))
    init_f = initialized.astype(jnp.float32).reshape(1, 1)

    v = pl.pallas_call(
        _v_body,
        out_shape=jax.ShapeDtypeStruct((1, DIM), jnp.float32),
    )(sums2, counts2, cent_p, init_f)

    out = pl.pallas_call(
        _proj_body,
        grid=(N // PROJ_BR,),
        in_specs=[pl.BlockSpec((PROJ_BR, DIM), lambda i: (i, 0)),
                  pl.BlockSpec((1, DIM), lambda i: (0, 0))],
        out_specs=pl.BlockSpec((PROJ_BR, DIM), lambda i: (i, 0)),
        out_shape=jax.ShapeDtypeStruct((N, DIM), jnp.float32),
    )(grads, v)
    return out


# speedup vs baseline: 7.2061x; 1.1455x over previous
import functools

import jax
import jax.numpy as jnp
from jax import lax
from jax.experimental import pallas as pl
from jax.experimental.pallas import tpu as pltpu
from jax.experimental.pallas import tpu_sc as plsc

N = 320000
DIM = 128
NUM_CLASSES = 1000
CP = 1024
MOMENTUM = 0.9

NC = 2
NS = 16
NW = NC * NS
BLK = 128
CHUNK = 256
NCHUNK = N // CHUNK
NSLOT = -(-NCHUNK // NW)
NSLOT += NSLOT % 2
ROWS_PER_TILE = CP // NS


def _seg_body(grads_hbm, labels_hbm, sums_hbm, counts_hbm,
              rows_a, rows_b, labs_a, labs_b, ones_v, zrow_v, acc_sh, cacc_sh,
              sem_l0, sem_l1, sem_s0, sem_s1):
    c = lax.axis_index("c")
    s = lax.axis_index("s")
    w = c * NS + s
    rows = (rows_a, rows_b)
    labs = (labs_a, labs_b)
    sem_l = (sem_l0, sem_l1)
    sem_s = (sem_s0, sem_s1)

    zero16 = jnp.zeros((16,), jnp.float32)
    one16 = jnp.ones((16,), jnp.float32)

    def fill_zero(r, _):
        for j in range(DIM // 16):
            zrow_v[r, pl.ds(j * 16, 16)] = zero16
        return 0

    lax.fori_loop(0, ROWS_PER_TILE, fill_zero, 0)

    def fill_ones(r, _):
        for j in range(DIM // 16):
            ones_v[r, pl.ds(j * 16, 16)] = one16
        return 0

    lax.fori_loop(0, BLK, fill_ones, 0)

    def cid_of(slot):
        return w + slot * NW

    def start_load(slot, b):
        @pl.when(cid_of(slot) < NCHUNK)
        def _():
            cid = cid_of(slot)
            pltpu.async_copy(grads_hbm.at[pl.ds(cid * CHUNK, CHUNK)],
                             rows[b], sem_l[b])
            for j in range(CHUNK // BLK):
                pltpu.async_copy(
                    labels_hbm.at[pl.ds(cid * CHUNK + j * BLK, BLK)],
                    labs[b].at[j], sem_l[b])

    def wait_load(b):
        pltpu.make_async_copy(grads_hbm.at[pl.ds(0, CHUNK)],
                              rows[b], sem_l[b]).wait()
        for j in range(CHUNK // BLK):
            pltpu.make_async_copy(labels_hbm.at[pl.ds(0, BLK)],
                                  labs[b].at[j], sem_l[b]).wait()

    def fire_scatters(b):
        for j in range(CHUNK // BLK):
            pltpu.async_copy(rows[b].at[pl.ds(j * BLK, BLK)],
                             acc_sh.at[labs[b].at[j]], sem_s[b], add=True)
            pltpu.async_copy(ones_v, cacc_sh.at[labs[b].at[j]],
                             sem_s[b], add=True)

    def wait_scatters(b):
        for j in range(CHUNK // BLK):
            pltpu.make_async_copy(rows[b].at[pl.ds(j * BLK, BLK)],
                                  acc_sh.at[labs[b].at[j]], sem_s[b]).wait()
            pltpu.make_async_copy(ones_v, cacc_sh.at[labs[b].at[j]],
                                  sem_s[b]).wait()

    start_load(0, 0)

    pltpu.sync_copy(zrow_v, acc_sh.at[pl.ds(s * ROWS_PER_TILE, ROWS_PER_TILE)])
    pltpu.sync_copy(zrow_v, cacc_sh.at[pl.ds(s * ROWS_PER_TILE, ROWS_PER_TILE)])
    plsc.subcore_barrier()

    def step(slot, b):
        valid = cid_of(slot) < NCHUNK

        @pl.when(valid)
        def _():
            wait_load(b)
            fire_scatters(b)

        @pl.when(jnp.logical_and(slot >= 1, cid_of(slot - 1) < NCHUNK))
        def _():
            wait_scatters(1 - b)
        start_load(slot + 1, 1 - b)

    def pair(p, _):
        step(2 * p, 0)
        step(2 * p + 1, 1)
        return 0

    lax.fori_loop(0, NSLOT // 2, pair, 0)

    @pl.when(cid_of(NSLOT - 1) < NCHUNK)
    def _():
        wait_scatters((NSLOT - 1) % 2)

    plsc.subcore_barrier()
    pltpu.sync_copy(acc_sh.at[pl.ds(s * ROWS_PER_TILE, ROWS_PER_TILE)],
                    sums_hbm.at[c, pl.ds(s * ROWS_PER_TILE, ROWS_PER_TILE)])
    pltpu.sync_copy(cacc_sh.at[pl.ds(s * ROWS_PER_TILE, ROWS_PER_TILE)],
                    counts_hbm.at[c, pl.ds(s * ROWS_PER_TILE, ROWS_PER_TILE)])


@functools.cache
def _build_seg_sum():
  return functools.partial(
    pl.kernel,
    out_type=(jax.ShapeDtypeStruct((NC, CP, DIM), jnp.float32),
              jax.ShapeDtypeStruct((NC, CP, DIM), jnp.float32)),
    mesh=plsc.VectorSubcoreMesh(core_axis_name="c", subcore_axis_name="s",
                                num_cores=NC, num_subcores=NS),
    scratch_types=[
        pltpu.VMEM((CHUNK, DIM), jnp.float32),
        pltpu.VMEM((CHUNK, DIM), jnp.float32),
        pltpu.VMEM((CHUNK // BLK, BLK), jnp.int32),
        pltpu.VMEM((CHUNK // BLK, BLK), jnp.int32),
        pltpu.VMEM((BLK, DIM), jnp.float32),
        pltpu.VMEM((ROWS_PER_TILE, DIM), jnp.float32),
        pltpu.VMEM_SHARED((CP, DIM), jnp.float32),
        pltpu.VMEM_SHARED((CP, DIM), jnp.float32),
        pltpu.SemaphoreType.DMA,
        pltpu.SemaphoreType.DMA,
        pltpu.SemaphoreType.DMA,
        pltpu.SemaphoreType.DMA,
    ],
  )(_seg_body)


def _compute_v(sums_ref, counts_ref, cent_ref, init_ref):
    sums = sums_ref[0] + sums_ref[1]
    counts = counts_ref[0, :, 0:1] + counts_ref[1, :, 0:1]
    means = sums / jnp.maximum(counts, 1.0)
    cent = cent_ref[...]
    init = init_ref[0, 0]
    ema = MOMENTUM * cent + (1.0 - MOMENTUM) * means
    upd = jnp.where(init > 0.5, ema, means)
    newc = jnp.where(counts > 0.0, upd, cent)
    rowid = lax.broadcasted_iota(jnp.int32, (CP, 1), 0)
    valid = rowid < NUM_CLASSES
    newc = jnp.where(valid, newc, 0.0)
    mean_c = jnp.sum(newc, axis=0, keepdims=True) * (1.0 / NUM_CLASSES)
    centered = jnp.where(valid, newc - mean_c, 0.0)

    gram = lax.dot_general(centered, centered, (((0,), (0,)), ((), ())),
                           preferred_element_type=jnp.float32,
                           precision=lax.Precision.HIGHEST)

    def _nrm(m):
        return m * lax.rsqrt(jnp.sum(m * m) + 1e-30)

    bmat = _nrm(gram)

    def _sq(_, m):
        return _nrm(jnp.dot(m, m, preferred_element_type=jnp.float32,
                            precision=lax.Precision.HIGHEST))

    bmat = lax.fori_loop(0, 18, _sq, bmat)

    ri = lax.broadcasted_iota(jnp.int32, (DIM, DIM), 0)
    ci = lax.broadcasted_iota(jnp.int32, (DIM, DIM), 1)
    diag = jnp.sum(jnp.where(ri == ci, bmat, 0.0), axis=1, keepdims=True)
    dmax = jnp.max(diag)
    v0 = jnp.sum(jnp.where(diag == dmax, bmat, 0.0), axis=0, keepdims=True)
    v0 = _nrm(v0)
    for _ in range(2):
        v0 = _nrm(jnp.dot(v0, gram, preferred_element_type=jnp.float32,
                          precision=lax.Precision.HIGHEST))
    return v0


def _proj_body(sums_ref, counts_ref, cent_ref, init_ref, g_ref, o_ref, v_s):
    @pl.when(pl.program_id(0) == 0)
    def _():
        v_s[...] = _compute_v(sums_ref, counts_ref, cent_ref, init_ref)

    g = g_ref[...]
    v = v_s[...]
    t = jnp.sum(g * v, axis=1, keepdims=True)
    o_ref[...] = g - t * v


PROJ_BR = 6400


def kernel(grads, labels, centroids, initialized):
    sums2, counts2 = _build_seg_sum()(grads, labels)

    cent_p = jnp.pad(centroids, ((0, CP - NUM_CLASSES), (0, 0)))
    init_f = initialized.astype(jnp.float32).reshape(1, 1)
    counts_c = counts2[:, :, :1]

    nsteps = N // PROJ_BR
    out = pl.pallas_call(
        _proj_body,
        grid=(nsteps,),
        in_specs=[
            pl.BlockSpec((NC, CP, DIM), lambda i: (0, 0, 0)),
            pl.BlockSpec((NC, CP, 1), lambda i: (0, 0, 0)),
            pl.BlockSpec((CP, DIM), lambda i: (0, 0)),
            pl.BlockSpec((1, 1), lambda i: (0, 0)),
            pl.BlockSpec((PROJ_BR, DIM), lambda i: (i, 0)),
        ],
        out_specs=pl.BlockSpec((PROJ_BR, DIM), lambda i: (i, 0)),
        out_shape=jax.ShapeDtypeStruct((N, DIM), jnp.float32),
        scratch_shapes=[pltpu.VMEM((1, DIM), jnp.float32)],
    )(sums2, counts_c, cent_p, init_f, grads)
    return out


# speedup vs baseline: 7.3862x; 1.0250x over previous
import functools

import jax
import jax.numpy as jnp
from jax import lax
from jax.experimental import pallas as pl
from jax.experimental.pallas import tpu as pltpu
from jax.experimental.pallas import tpu_sc as plsc

N = 320000
DIM = 128
NUM_CLASSES = 1000
CP = 1024
MOMENTUM = 0.9

NC = 2
NS = 16
NW = NC * NS
BLK = 128
CHUNK = 256
NCHUNK = N // CHUNK
NSLOT = -(-NCHUNK // NW)
NSLOT += NSLOT % 2
ROWS_PER_TILE = CP // NS


def _seg_body(grads_hbm, labels_hbm, sums_hbm, counts_hbm,
              rows_a, rows_b, labs_a, labs_b, ones_v, zrow_v, acc_sh, cacc_sh,
              sem_l0, sem_l1, sem_s0, sem_s1, sem_c):
    c = lax.axis_index("c")
    s = lax.axis_index("s")
    w = c * NS + s
    rows = (rows_a, rows_b)
    labs = (labs_a, labs_b)
    sem_l = (sem_l0, sem_l1)
    sem_s = (sem_s0, sem_s1)

    zero16 = jnp.zeros((16,), jnp.float32)
    one16 = jnp.ones((16,), jnp.float32)

    def fill_zero(r, _):
        for j in range(DIM // 16):
            zrow_v[r, pl.ds(j * 16, 16)] = zero16
        return 0

    lax.fori_loop(0, ROWS_PER_TILE, fill_zero, 0)

    def fill_ones(r, _):
        for j in range(DIM // 16):
            ones_v[r, pl.ds(j * 16, 16)] = one16
        return 0

    lax.fori_loop(0, BLK, fill_ones, 0)

    def cid_of(slot):
        return w + slot * NW

    def start_load(slot, b):
        @pl.when(cid_of(slot) < NCHUNK)
        def _():
            cid = cid_of(slot)
            pltpu.async_copy(grads_hbm.at[pl.ds(cid * CHUNK, CHUNK)],
                             rows[b], sem_l[b])
            for j in range(CHUNK // BLK):
                pltpu.async_copy(
                    labels_hbm.at[pl.ds(cid * CHUNK + j * BLK, BLK)],
                    labs[b].at[j], sem_l[b])

    def wait_load(b):
        pltpu.make_async_copy(grads_hbm.at[pl.ds(0, CHUNK)],
                              rows[b], sem_l[b]).wait()
        for j in range(CHUNK // BLK):
            pltpu.make_async_copy(labels_hbm.at[pl.ds(0, BLK)],
                                  labs[b].at[j], sem_l[b]).wait()

    def fire_scatters(b):
        for j in range(CHUNK // BLK):
            pltpu.async_copy(rows[b].at[pl.ds(j * BLK, BLK)],
                             acc_sh.at[labs[b].at[j]], sem_s[b], add=True)
            pltpu.async_copy(ones_v, cacc_sh.at[labs[b].at[j]],
                             sem_c, add=True)

    def wait_scatters(b):
        for j in range(CHUNK // BLK):
            pltpu.make_async_copy(rows[b].at[pl.ds(j * BLK, BLK)],
                                  acc_sh.at[labs[b].at[j]], sem_s[b]).wait()

    start_load(0, 0)

    pltpu.sync_copy(zrow_v, acc_sh.at[pl.ds(s * ROWS_PER_TILE, ROWS_PER_TILE)])
    pltpu.sync_copy(zrow_v, cacc_sh.at[pl.ds(s * ROWS_PER_TILE, ROWS_PER_TILE)])
    plsc.subcore_barrier()

    def step(slot, b):
        valid = cid_of(slot) < NCHUNK

        @pl.when(valid)
        def _():
            wait_load(b)
            fire_scatters(b)

        @pl.when(jnp.logical_and(slot >= 1, cid_of(slot - 1) < NCHUNK))
        def _():
            wait_scatters(1 - b)
        start_load(slot + 1, 1 - b)

    def pair(p, _):
        step(2 * p, 0)
        step(2 * p + 1, 1)
        return 0

    lax.fori_loop(0, NSLOT // 2, pair, 0)

    @pl.when(cid_of(NSLOT - 1) < NCHUNK)
    def _():
        wait_scatters((NSLOT - 1) % 2)

    def drain(i, _):
        @pl.when(cid_of(i) < NCHUNK)
        def _():
            for j in range(CHUNK // BLK):
                pltpu.make_async_copy(ones_v, cacc_sh.at[labs[0].at[j]],
                                      sem_c).wait()
        return 0

    lax.fori_loop(0, NSLOT, drain, 0)

    plsc.subcore_barrier()
    pltpu.sync_copy(acc_sh.at[pl.ds(s * ROWS_PER_TILE, ROWS_PER_TILE)],
                    sums_hbm.at[c, pl.ds(s * ROWS_PER_TILE, ROWS_PER_TILE)])
    pltpu.sync_copy(cacc_sh.at[pl.ds(s * ROWS_PER_TILE, ROWS_PER_TILE)],
                    counts_hbm.at[c, pl.ds(s * ROWS_PER_TILE, ROWS_PER_TILE)])


@functools.cache
def _build_seg_sum():
  return functools.partial(
    pl.kernel,
    out_type=(jax.ShapeDtypeStruct((NC, CP, DIM), jnp.float32),
              jax.ShapeDtypeStruct((NC, CP, DIM), jnp.float32)),
    mesh=plsc.VectorSubcoreMesh(core_axis_name="c", subcore_axis_name="s",
                                num_cores=NC, num_subcores=NS),
    scratch_types=[
        pltpu.VMEM((CHUNK, DIM), jnp.float32),
        pltpu.VMEM((CHUNK, DIM), jnp.float32),
        pltpu.VMEM((CHUNK // BLK, BLK), jnp.int32),
        pltpu.VMEM((CHUNK // BLK, BLK), jnp.int32),
        pltpu.VMEM((BLK, DIM), jnp.float32),
        pltpu.VMEM((ROWS_PER_TILE, DIM), jnp.float32),
        pltpu.VMEM_SHARED((CP, DIM), jnp.float32),
        pltpu.VMEM_SHARED((CP, DIM), jnp.float32),
        pltpu.SemaphoreType.DMA,
        pltpu.SemaphoreType.DMA,
        pltpu.SemaphoreType.DMA,
        pltpu.SemaphoreType.DMA,
        pltpu.SemaphoreType.DMA,
    ],
  )(_seg_body)


def _compute_v(sums_ref, counts_ref, cent_ref, init_ref):
    sums = sums_ref[0] + sums_ref[1]
    counts = counts_ref[0, :, 0:1] + counts_ref[1, :, 0:1]
    means = sums / jnp.maximum(counts, 1.0)
    cent = cent_ref[...]
    init = init_ref[0, 0]
    ema = MOMENTUM * cent + (1.0 - MOMENTUM) * means
    upd = jnp.where(init > 0.5, ema, means)
    newc = jnp.where(counts > 0.0, upd, cent)
    rowid = lax.broadcasted_iota(jnp.int32, (CP, 1), 0)
    valid = rowid < NUM_CLASSES
    newc = jnp.where(valid, newc, 0.0)
    mean_c = jnp.sum(newc, axis=0, keepdims=True) * (1.0 / NUM_CLASSES)
    centered = jnp.where(valid, newc - mean_c, 0.0)

    gram = lax.dot_general(centered, centered, (((0,), (0,)), ((), ())),
                           preferred_element_type=jnp.float32,
                           precision=lax.Precision.HIGHEST)

    def _nrm(m):
        return m * lax.rsqrt(jnp.sum(m * m) + 1e-30)

    bmat = _nrm(gram)

    def _sq(_, m):
        return _nrm(jnp.dot(m, m, preferred_element_type=jnp.float32,
                            precision=lax.Precision.HIGHEST))

    bmat = lax.fori_loop(0, 18, _sq, bmat)

    ri = lax.broadcasted_iota(jnp.int32, (DIM, DIM), 0)
    ci = lax.broadcasted_iota(jnp.int32, (DIM, DIM), 1)
    diag = jnp.sum(jnp.where(ri == ci, bmat, 0.0), axis=1, keepdims=True)
    dmax = jnp.max(diag)
    v0 = jnp.sum(jnp.where(diag == dmax, bmat, 0.0), axis=0, keepdims=True)
    v0 = _nrm(v0)
    for _ in range(2):
        v0 = _nrm(jnp.dot(v0, gram, preferred_element_type=jnp.float32,
                          precision=lax.Precision.HIGHEST))
    return v0


def _proj_body(sums_ref, counts_ref, cent_ref, init_ref, g_ref, o_ref, v_s):
    @pl.when(pl.program_id(0) == 0)
    def _():
        v_s[...] = _compute_v(sums_ref, counts_ref, cent_ref, init_ref)

    g = g_ref[...]
    v = v_s[...]
    t = jnp.sum(g * v, axis=1, keepdims=True)
    o_ref[...] = g - t * v


PROJ_BR = 12800


def kernel(grads, labels, centroids, initialized):
    sums2, counts2 = _build_seg_sum()(grads, labels)

    cent_p = jnp.pad(centroids, ((0, CP - NUM_CLASSES), (0, 0)))
    init_f = initialized.astype(jnp.float32).reshape(1, 1)
    counts_c = counts2[:, :, :1]

    nsteps = N // PROJ_BR
    out = pl.pallas_call(
        _proj_body,
        grid=(nsteps,),
        in_specs=[
            pl.BlockSpec((NC, CP, DIM), lambda i: (0, 0, 0)),
            pl.BlockSpec((NC, CP, 1), lambda i: (0, 0, 0)),
            pl.BlockSpec((CP, DIM), lambda i: (0, 0)),
            pl.BlockSpec((1, 1), lambda i: (0, 0)),
            pl.BlockSpec((PROJ_BR, DIM), lambda i: (i, 0)),
        ],
        out_specs=pl.BlockSpec((PROJ_BR, DIM), lambda i: (i, 0)),
        out_shape=jax.ShapeDtypeStruct((N, DIM), jnp.float32),
        scratch_shapes=[pltpu.VMEM((1, DIM), jnp.float32)],
    )(sums2, counts_c, cent_p, init_f, grads)
    return out


# speedup vs baseline: 7.4093x; 1.0031x over previous
import functools

import jax
import jax.numpy as jnp
from jax import lax
from jax.experimental import pallas as pl
from jax.experimental.pallas import tpu as pltpu
from jax.experimental.pallas import tpu_sc as plsc

N = 320000
DIM = 128
NUM_CLASSES = 1000
CP = 1024
MOMENTUM = 0.9

NC = 2
NS = 16
NW = NC * NS
BLK = 128
CHUNK = 256
NCHUNK = N // CHUNK
NSLOT = -(-NCHUNK // NW)
NSLOT += NSLOT % 2
ROWS_PER_TILE = CP // NS


def _seg_body(grads_hbm, labels_hbm, sums_hbm, counts_hbm,
              rows_a, rows_b, labs_a, labs_b, ones_v, zrow_v, acc_sh, cacc_sh,
              sem_l0, sem_l1, sem_s0, sem_s1, sem_c):
    c = lax.axis_index("c")
    s = lax.axis_index("s")
    w = c * NS + s
    rows = (rows_a, rows_b)
    labs = (labs_a, labs_b)
    sem_l = (sem_l0, sem_l1)
    sem_s = (sem_s0, sem_s1)

    zero16 = jnp.zeros((16,), jnp.float32)
    one16 = jnp.ones((16,), jnp.float32)

    def fill_zero(r, _):
        for j in range(DIM // 16):
            zrow_v[r, pl.ds(j * 16, 16)] = zero16
        return 0

    lax.fori_loop(0, ROWS_PER_TILE, fill_zero, 0)

    def fill_ones(r, _):
        for j in range(DIM // 16):
            ones_v[r, pl.ds(j * 16, 16)] = one16
        return 0

    lax.fori_loop(0, BLK, fill_ones, 0)

    def cid_of(slot):
        return w + slot * NW

    def start_load(slot, b):
        @pl.when(cid_of(slot) < NCHUNK)
        def _():
            cid = cid_of(slot)
            pltpu.async_copy(grads_hbm.at[pl.ds(cid * CHUNK, CHUNK)],
                             rows[b], sem_l[b])
            for j in range(CHUNK // BLK):
                pltpu.async_copy(
                    labels_hbm.at[pl.ds(cid * CHUNK + j * BLK, BLK)],
                    labs[b].at[j], sem_l[b])

    def wait_load(b):
        pltpu.make_async_copy(grads_hbm.at[pl.ds(0, CHUNK)],
                              rows[b], sem_l[b]).wait()
        for j in range(CHUNK // BLK):
            pltpu.make_async_copy(labels_hbm.at[pl.ds(0, BLK)],
                                  labs[b].at[j], sem_l[b]).wait()

    def fire_scatters(b):
        for j in range(CHUNK // BLK):
            pltpu.async_copy(rows[b].at[pl.ds(j * BLK, BLK)],
                             acc_sh.at[labs[b].at[j]], sem_s[b], add=True)
            pltpu.async_copy(ones_v, cacc_sh.at[labs[b].at[j]],
                             sem_c, add=True)

    def wait_scatters(b):
        for j in range(CHUNK // BLK):
            pltpu.make_async_copy(rows[b].at[pl.ds(j * BLK, BLK)],
                                  acc_sh.at[labs[b].at[j]], sem_s[b]).wait()

    start_load(0, 0)

    pltpu.sync_copy(zrow_v, acc_sh.at[pl.ds(s * ROWS_PER_TILE, ROWS_PER_TILE)])
    pltpu.sync_copy(zrow_v, cacc_sh.at[pl.ds(s * ROWS_PER_TILE, ROWS_PER_TILE)])
    plsc.subcore_barrier()

    def step(slot, b):
        valid = cid_of(slot) < NCHUNK

        @pl.when(valid)
        def _():
            wait_load(b)
            fire_scatters(b)

        @pl.when(jnp.logical_and(slot >= 1, cid_of(slot - 1) < NCHUNK))
        def _():
            wait_scatters(1 - b)
        start_load(slot + 1, 1 - b)

    def pair(p, _):
        step(2 * p, 0)
        step(2 * p + 1, 1)
        return 0

    lax.fori_loop(0, NSLOT // 2, pair, 0)

    @pl.when(cid_of(NSLOT - 1) < NCHUNK)
    def _():
        wait_scatters((NSLOT - 1) % 2)

    def drain(i, _):
        @pl.when(cid_of(i) < NCHUNK)
        def _():
            for j in range(CHUNK // BLK):
                pltpu.make_async_copy(ones_v, cacc_sh.at[labs[0].at[j]],
                                      sem_c).wait()
        return 0

    lax.fori_loop(0, NSLOT, drain, 0)

    plsc.subcore_barrier()
    pltpu.sync_copy(acc_sh.at[pl.ds(s * ROWS_PER_TILE, ROWS_PER_TILE)],
                    sums_hbm.at[c, pl.ds(s * ROWS_PER_TILE, ROWS_PER_TILE)])
    pltpu.sync_copy(cacc_sh.at[pl.ds(s * ROWS_PER_TILE, ROWS_PER_TILE)],
                    counts_hbm.at[c, pl.ds(s * ROWS_PER_TILE, ROWS_PER_TILE)])


@functools.cache
def _build_seg_sum():
  return functools.partial(
    pl.kernel,
    out_type=(jax.ShapeDtypeStruct((NC, CP, DIM), jnp.float32),
              jax.ShapeDtypeStruct((NC, CP, DIM), jnp.float32)),
    mesh=plsc.VectorSubcoreMesh(core_axis_name="c", subcore_axis_name="s",
                                num_cores=NC, num_subcores=NS),
    scratch_types=[
        pltpu.VMEM((CHUNK, DIM), jnp.float32),
        pltpu.VMEM((CHUNK, DIM), jnp.float32),
        pltpu.VMEM((CHUNK // BLK, BLK), jnp.int32),
        pltpu.VMEM((CHUNK // BLK, BLK), jnp.int32),
        pltpu.VMEM((BLK, DIM), jnp.float32),
        pltpu.VMEM((ROWS_PER_TILE, DIM), jnp.float32),
        pltpu.VMEM_SHARED((CP, DIM), jnp.float32),
        pltpu.VMEM_SHARED((CP, DIM), jnp.float32),
        pltpu.SemaphoreType.DMA,
        pltpu.SemaphoreType.DMA,
        pltpu.SemaphoreType.DMA,
        pltpu.SemaphoreType.DMA,
        pltpu.SemaphoreType.DMA,
    ],
  )(_seg_body)


def _compute_v(sums_ref, counts_ref, cent_ref, init_ref):
    sums = sums_ref[0] + sums_ref[1]
    counts = counts_ref[0, :, 0:1] + counts_ref[1, :, 0:1]
    means = sums / jnp.maximum(counts, 1.0)
    cent = cent_ref[...]
    init = init_ref[0, 0]
    ema = MOMENTUM * cent + (1.0 - MOMENTUM) * means
    upd = jnp.where(init > 0.5, ema, means)
    newc = jnp.where(counts > 0.0, upd, cent)
    rowid = lax.broadcasted_iota(jnp.int32, (CP, 1), 0)
    valid = rowid < NUM_CLASSES
    newc = jnp.where(valid, newc, 0.0)
    mean_c = jnp.sum(newc, axis=0, keepdims=True) * (1.0 / NUM_CLASSES)
    centered = jnp.where(valid, newc - mean_c, 0.0)

    gram = lax.dot_general(centered, centered, (((0,), (0,)), ((), ())),
                           preferred_element_type=jnp.float32,
                           precision=lax.Precision.HIGHEST)

    def _nrm(m):
        return m * lax.rsqrt(jnp.sum(m * m) + 1e-30)

    bmat = _nrm(gram)

    def _sq(_, m):
        return _nrm(jnp.dot(m, m, preferred_element_type=jnp.float32,
                            precision=lax.Precision.HIGHEST))

    bmat = lax.fori_loop(0, 18, _sq, bmat)

    ri = lax.broadcasted_iota(jnp.int32, (DIM, DIM), 0)
    ci = lax.broadcasted_iota(jnp.int32, (DIM, DIM), 1)
    diag = jnp.sum(jnp.where(ri == ci, bmat, 0.0), axis=1, keepdims=True)
    dmax = jnp.max(diag)
    v0 = jnp.sum(jnp.where(diag == dmax, bmat, 0.0), axis=0, keepdims=True)
    v0 = _nrm(v0)
    for _ in range(2):
        v0 = _nrm(jnp.dot(v0, gram, preferred_element_type=jnp.float32,
                          precision=lax.Precision.HIGHEST))
    return v0


def _proj_body(sums_ref, counts_ref, cent_ref, init_ref, g_ref, o_ref, v_s):
    @pl.when(pl.program_id(0) == 0)
    def _():
        v_s[...] = _compute_v(sums_ref, counts_ref, cent_ref, init_ref)

    g = g_ref[...]
    v = v_s[...]
    t = jnp.sum(g * v, axis=1, keepdims=True)
    o_ref[...] = g - t * v


PROJ_BR = 16000


def kernel(grads, labels, centroids, initialized):
    sums2, counts2 = _build_seg_sum()(grads, labels)

    cent_p = jnp.pad(centroids, ((0, CP - NUM_CLASSES), (0, 0)))
    init_f = initialized.astype(jnp.float32).reshape(1, 1)
    counts_c = counts2[:, :, :1]

    nsteps = N // PROJ_BR
    out = pl.pallas_call(
        _proj_body,
        grid=(nsteps,),
        in_specs=[
            pl.BlockSpec((NC, CP, DIM), lambda i: (0, 0, 0)),
            pl.BlockSpec((NC, CP, 1), lambda i: (0, 0, 0)),
            pl.BlockSpec((CP, DIM), lambda i: (0, 0)),
            pl.BlockSpec((1, 1), lambda i: (0, 0)),
            pl.BlockSpec((PROJ_BR, DIM), lambda i: (i, 0)),
        ],
        out_specs=pl.BlockSpec((PROJ_BR, DIM), lambda i: (i, 0)),
        out_shape=jax.ShapeDtypeStruct((N, DIM), jnp.float32),
        scratch_shapes=[pltpu.VMEM((1, DIM), jnp.float32)],
    )(sums2, counts_c, cent_p, init_f, grads)
    return out
